# Initial kernel scaffold; baseline (speedup 1.0000x reference)
import jax
import jax.numpy as jnp
from jax.experimental import pallas as pl


def kernel(features, edge_index, W1, b1, W2, b2, W3, b3):
    raise NotImplementedError("write your pallas kernel here")


# speedup vs baseline: 6.2685x; 6.2685x over previous
import functools

import jax
import jax.numpy as jnp
from jax import lax
from jax.experimental import pallas as pl
from jax.experimental.pallas import tpu as pltpu
from jax.experimental.pallas import tpu_sc as plsc

N = 10000
E = 320000
D_H = 128
N_CLS = 40
D3 = 64

NC = 2
NS = 16
NW = NC * NS
EPW = E // NW
K = 125
NCH = EPW // K
N_P = 10240
RPS = N_P // NS

_MESH = plsc.VectorSubcoreMesh(core_axis_name="c", subcore_axis_name="s",
                               num_cores=NC, num_subcores=NS)


def _hist_body(srcr_hbm, dstr_hbm, zeros_hbm, ones_hbm,
               hs_out, hd_out, idx_s, idx_d, ones_v, acc):
    c = lax.axis_index("c")
    s = lax.axis_index("s")
    wid = c * NS + s
    pltpu.sync_copy(srcr_hbm.at[wid], idx_s)
    pltpu.sync_copy(dstr_hbm.at[wid], idx_d)
    pltpu.sync_copy(ones_hbm, ones_v)
    ones_k = ones_v.at[pl.ds(0, K)]
    sl = pl.ds(s * RPS, RPS)

    for idx, out in ((idx_s, hs_out), (idx_d, hd_out)):
        pltpu.sync_copy(zeros_hbm.at[sl], acc.at[sl])
        plsc.subcore_barrier()

        @pl.loop(0, NCH)
        def _(j, idx=idx):
            pltpu.sync_copy(ones_k, acc.at[idx.at[j]], add=True)

        plsc.subcore_barrier()
        pltpu.sync_copy(acc.at[sl], out.at[c].at[sl])
        plsc.subcore_barrier()


@jax.jit
def _sc_hist(src_r, dst_r, zeros128, ones128):
    f32 = jnp.float32
    return pl.kernel(
        _hist_body,
        out_type=(jax.ShapeDtypeStruct((NC, N_P, D_H), f32),
                  jax.ShapeDtypeStruct((NC, N_P, D_H), f32)),
        mesh=_MESH,
        scratch_types=[
            pltpu.VMEM((NCH, K), jnp.int32),
            pltpu.VMEM((NCH, K), jnp.int32),
            pltpu.VMEM((128, D_H), f32),
            pltpu.VMEM_SHARED((N_P, D_H), f32),
        ],
    )(src_r, dst_r, zeros128, ones128)


def _agg_body(y_hbm, srcr_hbm, dstr_hbm, zeros_hbm,
              out_hbm, idx_s, idx_d, rows, acc):
    c = lax.axis_index("c")
    s = lax.axis_index("s")
    wid = c * NS + s
    pltpu.sync_copy(srcr_hbm.at[wid], idx_s)
    pltpu.sync_copy(dstr_hbm.at[wid], idx_d)
    sl = pl.ds(s * RPS, RPS)
    pltpu.sync_copy(zeros_hbm.at[sl], acc.at[sl])
    plsc.subcore_barrier()

    @pl.loop(0, NCH)
    def _(j):
        pltpu.sync_copy(y_hbm.at[idx_s.at[j]], rows)
        pltpu.sync_copy(rows, acc.at[idx_d.at[j]], add=True)

    plsc.subcore_barrier()
    pltpu.sync_copy(acc.at[sl], out_hbm.at[c].at[sl])


@functools.partial(jax.jit, static_argnames=("d",))
def _sc_agg(y, src_r, dst_r, zeros, *, d):
    f32 = jnp.float32
    return pl.kernel(
        _agg_body,
        out_type=jax.ShapeDtypeStruct((NC, N_P, d), f32),
        mesh=_MESH,
        scratch_types=[
            pltpu.VMEM((NCH, K), jnp.int32),
            pltpu.VMEM((NCH, K), jnp.int32),
            pltpu.VMEM((K, d), f32),
            pltpu.VMEM_SHARED((N_P, d), f32),
        ],
    )(y, src_r, dst_r, zeros)


_BLK = 1000
_HIGHEST = jax.lax.Precision.HIGHEST


def _tc_first_body(f_ref, hs0_ref, hs1_ref, w_ref, o_ref):
    deg = hs0_ref[:, :1] + hs1_ref[:, :1]
    ns = lax.rsqrt(jnp.maximum(deg, 1.0))
    o_ref[...] = jnp.dot(f_ref[...] * ns, w_ref[...],
                         preferred_element_type=jnp.float32,
                         precision=_HIGHEST)


def _tc_mid_body(a0_ref, a1_ref, hd0_ref, hd1_ref, hs0_ref, hs1_ref,
                 b_ref, w_ref, o_ref):
    degd = hd0_ref[:, :1] + hd1_ref[:, :1]
    nd = lax.rsqrt(jnp.maximum(degd, 1.0))
    degs = hs0_ref[:, :1] + hs1_ref[:, :1]
    ns = lax.rsqrt(jnp.maximum(degs, 1.0))
    h = jnp.maximum((a0_ref[...] + a1_ref[...]) * nd + b_ref[...], 0.0)
    o_ref[...] = jnp.dot(h * ns, w_ref[...],
                         preferred_element_type=jnp.float32,
                         precision=_HIGHEST)


def _tc_prefinal_body(a0_ref, a1_ref, hd0_ref, hd1_ref, hs0_ref, hs1_ref,
                      b_ref, o_ref):
    degd = hd0_ref[:, :1] + hd1_ref[:, :1]
    nd = lax.rsqrt(jnp.maximum(degd, 1.0))
    degs = hs0_ref[:, :1] + hs1_ref[:, :1]
    ns = lax.rsqrt(jnp.maximum(degs, 1.0))
    h = jnp.maximum((a0_ref[...] + a1_ref[...]) * nd + b_ref[...], 0.0)
    o_ref[...] = h * ns


def _tc_final_body(a0_ref, a1_ref, hd0_ref, hd1_ref, b_ref, w_ref, o_ref):
    degd = hd0_ref[:, :1] + hd1_ref[:, :1]
    nd = lax.rsqrt(jnp.maximum(degd, 1.0))
    agg = (a0_ref[...] + a1_ref[...]) * nd
    o_ref[...] = jnp.dot(agg, w_ref[...],
                         preferred_element_type=jnp.float32,
                         precision=_HIGHEST) + b_ref[...]


def _rows(d):
    return pl.BlockSpec((_BLK, d), lambda i: (i, 0))


def _whole(shape):
    return pl.BlockSpec(shape, lambda i: (0,) * len(shape))


def _tc_first(features, hs0, hs1, W):
    dout = W.shape[1]
    return pl.pallas_call(
        _tc_first_body,
        grid=(N // _BLK,),
        in_specs=[_rows(D_H), _rows(D_H), _rows(D_H), _whole((D_H, dout))],
        out_specs=_rows(dout),
        out_shape=jax.ShapeDtypeStruct((N, dout), jnp.float32),
    )(features, hs0, hs1, W)


def _tc_mid(a0, a1, hd0, hd1, hs0, hs1, b, W):
    din, dout = W.shape
    return pl.pallas_call(
        _tc_mid_body,
        grid=(N // _BLK,),
        in_specs=[_rows(din), _rows(din), _rows(D_H), _rows(D_H),
                  _rows(D_H), _rows(D_H), _whole((1, din)), _whole((din, dout))],
        out_specs=_rows(dout),
        out_shape=jax.ShapeDtypeStruct((N, dout), jnp.float32),
    )(a0, a1, hd0, hd1, hs0, hs1, b, W)


def _tc_prefinal(a0, a1, hd0, hd1, hs0, hs1, b):
    d = a0.shape[1]
    return pl.pallas_call(
        _tc_prefinal_body,
        grid=(N // _BLK,),
        in_specs=[_rows(d), _rows(d), _rows(D_H), _rows(D_H),
                  _rows(D_H), _rows(D_H), _whole((1, d))],
        out_specs=_rows(d),
        out_shape=jax.ShapeDtypeStruct((N, d), jnp.float32),
    )(a0, a1, hd0, hd1, hs0, hs1, b)


def _tc_final(a0, a1, hd0, hd1, b, W):
    din, dout = W.shape
    return pl.pallas_call(
        _tc_final_body,
        grid=(N // _BLK,),
        in_specs=[_rows(din), _rows(din), _rows(D_H), _rows(D_H),
                  _whole((1, dout)), _whole((din, dout))],
        out_specs=_rows(dout),
        out_shape=jax.ShapeDtypeStruct((N, dout), jnp.float32),
    )(a0, a1, hd0, hd1, b, W)


def kernel(features, edge_index, W1, b1, W2, b2, W3, b3):
    f32 = jnp.float32
    src_r = edge_index[0].reshape(NW, NCH, K)
    dst_r = edge_index[1].reshape(NW, NCH, K)
    zeros128 = jnp.zeros((N_P, D_H), f32)
    ones128 = jnp.ones((128, D_H), f32)

    hs, hd = _sc_hist(src_r, dst_r, zeros128, ones128)
    hs0, hs1 = hs[0], hs[1]
    hd0, hd1 = hd[0], hd[1]

    y1 = _tc_first(features, hs0, hs1, W1)
    a1 = _sc_agg(y1, src_r, dst_r, zeros128, d=D_H)
    y2 = _tc_mid(a1[0], a1[1], hd0, hd1, hs0, hs1, b1.reshape(1, D_H), W2)
    a2 = _sc_agg(y2, src_r, dst_r, zeros128, d=D_H)
    h2 = _tc_prefinal(a2[0], a2[1], hd0, hd1, hs0, hs1, b2.reshape(1, D_H))
    a3 = _sc_agg(h2, src_r, dst_r, zeros128, d=D_H)
    return _tc_final(a3[0], a3[1], hd0, hd1, b3.reshape(1, N_CLS), W3)


# speedup vs baseline: 7.0552x; 1.1255x over previous
import functools

import jax
import jax.numpy as jnp
from jax import lax
from jax.experimental import pallas as pl
from jax.experimental.pallas import tpu as pltpu
from jax.experimental.pallas import tpu_sc as plsc

N = 10000
E = 320000
D_H = 128
N_CLS = 40
D3 = 64

NC = 2
NS = 16
NW = NC * NS
EPW = E // NW
K = 125
NCH = EPW // K
N_P = 10240
RPS = N_P // NS

_MESH = plsc.VectorSubcoreMesh(core_axis_name="c", subcore_axis_name="s",
                               num_cores=NC, num_subcores=NS)


def _hist_body(srcr_hbm, dstr_hbm, zeros_hbm, ones_hbm,
               hs_out, hd_out, idx_s, idx_d, ones_v, acc):
    c = lax.axis_index("c")
    s = lax.axis_index("s")
    wid = c * NS + s
    pltpu.sync_copy(srcr_hbm.at[wid], idx_s)
    pltpu.sync_copy(dstr_hbm.at[wid], idx_d)
    pltpu.sync_copy(ones_hbm, ones_v)
    ones_k = ones_v.at[pl.ds(0, K)]
    sl = pl.ds(s * RPS, RPS)

    for idx, out in ((idx_s, hs_out), (idx_d, hd_out)):
        pltpu.sync_copy(zeros_hbm.at[sl], acc.at[sl])
        plsc.subcore_barrier()

        @pl.loop(0, NCH)
        def _(j, idx=idx):
            pltpu.sync_copy(ones_k, acc.at[idx.at[j]], add=True)

        plsc.subcore_barrier()
        pltpu.sync_copy(acc.at[sl], out.at[c].at[sl])
        plsc.subcore_barrier()


@jax.jit
def _sc_hist(src_r, dst_r, zeros128, ones128):
    f32 = jnp.float32
    return pl.kernel(
        _hist_body,
        out_type=(jax.ShapeDtypeStruct((NC, N_P, D_H), f32),
                  jax.ShapeDtypeStruct((NC, N_P, D_H), f32)),
        mesh=_MESH,
        scratch_types=[
            pltpu.VMEM((NCH, K), jnp.int32),
            pltpu.VMEM((NCH, K), jnp.int32),
            pltpu.VMEM((128, D_H), f32),
            pltpu.VMEM_SHARED((N_P, D_H), f32),
        ],
    )(src_r, dst_r, zeros128, ones128)


HCH = NCH // 2


def _agg_body(y_hbm, srcr_hbm, dstr_hbm, zeros_hbm,
              out_hbm, idx_s, idx_d, rows, acc, sem0, sem1):
    c = lax.axis_index("c")
    s = lax.axis_index("s")
    wid = c * NS + s
    sl = pl.ds(s * RPS, RPS)
    pltpu.sync_copy(zeros_hbm.at[sl], acc.at[sl])
    plsc.subcore_barrier()

    r0 = rows.at[0]
    r1 = rows.at[1]
    for h in range(2):
        hsl = pl.ds(h * HCH, HCH)
        pltpu.sync_copy(srcr_hbm.at[wid].at[hsl], idx_s)
        pltpu.sync_copy(dstr_hbm.at[wid].at[hsl], idx_d)
        pltpu.async_copy(y_hbm.at[idx_s.at[0]], r0, sem0)
        pltpu.async_copy(y_hbm.at[idx_s.at[1]], r1, sem1)

        @pl.loop(0, HCH - 2, step=2)
        def _(j):
            pltpu.make_async_copy(y_hbm.at[idx_s.at[j]], r0, sem0).wait()
            pltpu.async_copy(r0, acc.at[idx_d.at[j]], sem0, add=True)
            pltpu.make_async_copy(y_hbm.at[idx_s.at[j + 1]], r1, sem1).wait()
            pltpu.async_copy(r1, acc.at[idx_d.at[j + 1]], sem1, add=True)
            pltpu.make_async_copy(r0, acc.at[idx_d.at[j]], sem0).wait()
            pltpu.async_copy(y_hbm.at[idx_s.at[j + 2]], r0, sem0)
            pltpu.make_async_copy(r1, acc.at[idx_d.at[j + 1]], sem1).wait()
            pltpu.async_copy(y_hbm.at[idx_s.at[j + 3]], r1, sem1)

        j0 = HCH - 2
        j1 = HCH - 1
        pltpu.make_async_copy(y_hbm.at[idx_s.at[j0]], r0, sem0).wait()
        pltpu.async_copy(r0, acc.at[idx_d.at[j0]], sem0, add=True)
        pltpu.make_async_copy(y_hbm.at[idx_s.at[j1]], r1, sem1).wait()
        pltpu.async_copy(r1, acc.at[idx_d.at[j1]], sem1, add=True)
        pltpu.make_async_copy(r0, acc.at[idx_d.at[j0]], sem0).wait()
        pltpu.make_async_copy(r1, acc.at[idx_d.at[j1]], sem1).wait()

    plsc.subcore_barrier()
    pltpu.sync_copy(acc.at[sl], out_hbm.at[c].at[sl])


@functools.partial(jax.jit, static_argnames=("d",))
def _sc_agg(y, src_r, dst_r, zeros, *, d):
    f32 = jnp.float32
    return pl.kernel(
        _agg_body,
        out_type=jax.ShapeDtypeStruct((NC, N_P, d), f32),
        mesh=_MESH,
        scratch_types=[
            pltpu.VMEM((HCH, K), jnp.int32),
            pltpu.VMEM((HCH, K), jnp.int32),
            pltpu.VMEM((2, K, d), f32),
            pltpu.VMEM_SHARED((N_P, d), f32),
            pltpu.SemaphoreType.DMA,
            pltpu.SemaphoreType.DMA,
        ],
    )(y, src_r, dst_r, zeros)


_BLK = 1000
_HIGHEST = jax.lax.Precision.HIGHEST


def _tc_first_body(f_ref, hs0_ref, hs1_ref, w_ref, o_ref):
    deg = hs0_ref[:, :1] + hs1_ref[:, :1]
    ns = lax.rsqrt(jnp.maximum(deg, 1.0))
    o_ref[...] = jnp.dot(f_ref[...] * ns, w_ref[...],
                         preferred_element_type=jnp.float32,
                         precision=_HIGHEST)


def _tc_mid_body(a0_ref, a1_ref, hd0_ref, hd1_ref, hs0_ref, hs1_ref,
                 b_ref, w_ref, o_ref):
    degd = hd0_ref[:, :1] + hd1_ref[:, :1]
    nd = lax.rsqrt(jnp.maximum(degd, 1.0))
    degs = hs0_ref[:, :1] + hs1_ref[:, :1]
    ns = lax.rsqrt(jnp.maximum(degs, 1.0))
    h = jnp.maximum((a0_ref[...] + a1_ref[...]) * nd + b_ref[...], 0.0)
    o_ref[...] = jnp.dot(h * ns, w_ref[...],
                         preferred_element_type=jnp.float32,
                         precision=_HIGHEST)


def _tc_prefinal_body(a0_ref, a1_ref, hd0_ref, hd1_ref, hs0_ref, hs1_ref,
                      b_ref, o_ref):
    degd = hd0_ref[:, :1] + hd1_ref[:, :1]
    nd = lax.rsqrt(jnp.maximum(degd, 1.0))
    degs = hs0_ref[:, :1] + hs1_ref[:, :1]
    ns = lax.rsqrt(jnp.maximum(degs, 1.0))
    h = jnp.maximum((a0_ref[...] + a1_ref[...]) * nd + b_ref[...], 0.0)
    o_ref[...] = h * ns


def _tc_final_body(a0_ref, a1_ref, hd0_ref, hd1_ref, b_ref, w_ref, o_ref):
    degd = hd0_ref[:, :1] + hd1_ref[:, :1]
    nd = lax.rsqrt(jnp.maximum(degd, 1.0))
    agg = (a0_ref[...] + a1_ref[...]) * nd
    o_ref[...] = jnp.dot(agg, w_ref[...],
                         preferred_element_type=jnp.float32,
                         precision=_HIGHEST) + b_ref[...]


def _rows(d):
    return pl.BlockSpec((_BLK, d), lambda i: (i, 0))


def _whole(shape):
    return pl.BlockSpec(shape, lambda i: (0,) * len(shape))


def _tc_first(features, hs0, hs1, W):
    dout = W.shape[1]
    return pl.pallas_call(
        _tc_first_body,
        grid=(N // _BLK,),
        in_specs=[_rows(D_H), _rows(D_H), _rows(D_H), _whole((D_H, dout))],
        out_specs=_rows(dout),
        out_shape=jax.ShapeDtypeStruct((N, dout), jnp.float32),
    )(features, hs0, hs1, W)


def _tc_mid(a0, a1, hd0, hd1, hs0, hs1, b, W):
    din, dout = W.shape
    return pl.pallas_call(
        _tc_mid_body,
        grid=(N // _BLK,),
        in_specs=[_rows(din), _rows(din), _rows(D_H), _rows(D_H),
                  _rows(D_H), _rows(D_H), _whole((1, din)), _whole((din, dout))],
        out_specs=_rows(dout),
        out_shape=jax.ShapeDtypeStruct((N, dout), jnp.float32),
    )(a0, a1, hd0, hd1, hs0, hs1, b, W)


def _tc_prefinal(a0, a1, hd0, hd1, hs0, hs1, b):
    d = a0.shape[1]
    return pl.pallas_call(
        _tc_prefinal_body,
        grid=(N // _BLK,),
        in_specs=[_rows(d), _rows(d), _rows(D_H), _rows(D_H),
                  _rows(D_H), _rows(D_H), _whole((1, d))],
        out_specs=_rows(d),
        out_shape=jax.ShapeDtypeStruct((N, d), jnp.float32),
    )(a0, a1, hd0, hd1, hs0, hs1, b)


def _tc_final(a0, a1, hd0, hd1, b, W):
    din, dout = W.shape
    return pl.pallas_call(
        _tc_final_body,
        grid=(N // _BLK,),
        in_specs=[_rows(din), _rows(din), _rows(D_H), _rows(D_H),
                  _whole((1, dout)), _whole((din, dout))],
        out_specs=_rows(dout),
        out_shape=jax.ShapeDtypeStruct((N, dout), jnp.float32),
    )(a0, a1, hd0, hd1, b, W)


def kernel(features, edge_index, W1, b1, W2, b2, W3, b3):
    f32 = jnp.float32
    src_r = edge_index[0].reshape(NW, NCH, K)
    dst_r = edge_index[1].reshape(NW, NCH, K)
    zeros128 = jnp.zeros((N_P, D_H), f32)
    ones128 = jnp.ones((128, D_H), f32)

    hs, hd = _sc_hist(src_r, dst_r, zeros128, ones128)
    hs0, hs1 = hs[0], hs[1]
    hd0, hd1 = hd[0], hd[1]

    y1 = _tc_first(features, hs0, hs1, W1)
    a1 = _sc_agg(y1, src_r, dst_r, zeros128, d=D_H)
    y2 = _tc_mid(a1[0], a1[1], hd0, hd1, hs0, hs1, b1.reshape(1, D_H), W2)
    a2 = _sc_agg(y2, src_r, dst_r, zeros128, d=D_H)
    h2 = _tc_prefinal(a2[0], a2[1], hd0, hd1, hs0, hs1, b2.reshape(1, D_H))
    a3 = _sc_agg(h2, src_r, dst_r, zeros128, d=D_H)
    return _tc_final(a3[0], a3[1], hd0, hd1, b3.reshape(1, N_CLS), W3)


# speedup vs baseline: 7.0615x; 1.0009x over previous
import functools

import jax
import jax.numpy as jnp
from jax import lax
from jax.experimental import pallas as pl
from jax.experimental.pallas import tpu as pltpu
from jax.experimental.pallas import tpu_sc as plsc

N = 10000
E = 320000
D_H = 128
N_CLS = 40
D3 = 64

NC = 2
NS = 16
NW = NC * NS
EPW = E // NW
K = 125
NCH = EPW // K
N_P = 10240
RPS = N_P // NS

_MESH = plsc.VectorSubcoreMesh(core_axis_name="c", subcore_axis_name="s",
                               num_cores=NC, num_subcores=NS)


def _hist_body(srcr_hbm, dstr_hbm, zeros_hbm, ones_hbm,
               hs_out, hd_out, idx_s, idx_d, ones_v, acc,
               sem0, sem1, sem2, sem3):
    c = lax.axis_index("c")
    s = lax.axis_index("s")
    wid = c * NS + s
    pltpu.sync_copy(srcr_hbm.at[wid], idx_s)
    pltpu.sync_copy(dstr_hbm.at[wid], idx_d)
    pltpu.sync_copy(ones_hbm, ones_v)
    ones_k = ones_v.at[pl.ds(0, K)]
    sl = pl.ds(s * RPS, RPS)
    sems = (sem0, sem1, sem2, sem3)

    for idx, out in ((idx_s, hs_out), (idx_d, hd_out)):
        pltpu.sync_copy(zeros_hbm.at[sl], acc.at[sl])
        plsc.subcore_barrier()

        for k in range(4):
            pltpu.async_copy(ones_k, acc.at[idx.at[k]], sems[k], add=True)

        @pl.loop(0, NCH - 4, step=4)
        def _(j, idx=idx):
            for k in range(4):
                pltpu.make_async_copy(ones_k, acc.at[idx.at[j + k]],
                                      sems[k]).wait()
                pltpu.async_copy(ones_k, acc.at[idx.at[j + 4 + k]],
                                 sems[k], add=True)

        for k in range(4):
            pltpu.make_async_copy(ones_k, acc.at[idx.at[NCH - 4 + k]],
                                  sems[k]).wait()

        plsc.subcore_barrier()
        pltpu.sync_copy(acc.at[sl], out.at[c].at[sl])
        plsc.subcore_barrier()


@jax.jit
def _sc_hist(src_r, dst_r, zeros128, ones128):
    f32 = jnp.float32
    return pl.kernel(
        _hist_body,
        out_type=(jax.ShapeDtypeStruct((NC, N_P, D_H), f32),
                  jax.ShapeDtypeStruct((NC, N_P, D_H), f32)),
        mesh=_MESH,
        scratch_types=[
            pltpu.VMEM((NCH, K), jnp.int32),
            pltpu.VMEM((NCH, K), jnp.int32),
            pltpu.VMEM((128, D_H), f32),
            pltpu.VMEM_SHARED((N_P, D_H), f32),
            pltpu.SemaphoreType.DMA,
            pltpu.SemaphoreType.DMA,
            pltpu.SemaphoreType.DMA,
            pltpu.SemaphoreType.DMA,
        ],
    )(src_r, dst_r, zeros128, ones128)


HCH = NCH // 2


def _agg_body(y_hbm, srcr_hbm, dstr_hbm, zeros_hbm,
              out_hbm, idx_s, idx_d, rows, acc, sem0, sem1):
    c = lax.axis_index("c")
    s = lax.axis_index("s")
    wid = c * NS + s
    sl = pl.ds(s * RPS, RPS)
    pltpu.sync_copy(zeros_hbm.at[sl], acc.at[sl])
    plsc.subcore_barrier()

    r0 = rows.at[0]
    r1 = rows.at[1]
    for h in range(2):
        hsl = pl.ds(h * HCH, HCH)
        pltpu.sync_copy(srcr_hbm.at[wid].at[hsl], idx_s)
        pltpu.sync_copy(dstr_hbm.at[wid].at[hsl], idx_d)
        pltpu.async_copy(y_hbm.at[idx_s.at[0]], r0, sem0)
        pltpu.async_copy(y_hbm.at[idx_s.at[1]], r1, sem1)

        @pl.loop(0, HCH - 2, step=2)
        def _(j):
            pltpu.make_async_copy(y_hbm.at[idx_s.at[j]], r0, sem0).wait()
            pltpu.async_copy(r0, acc.at[idx_d.at[j]], sem0, add=True)
            pltpu.make_async_copy(y_hbm.at[idx_s.at[j + 1]], r1, sem1).wait()
            pltpu.async_copy(r1, acc.at[idx_d.at[j + 1]], sem1, add=True)
            pltpu.make_async_copy(r0, acc.at[idx_d.at[j]], sem0).wait()
            pltpu.async_copy(y_hbm.at[idx_s.at[j + 2]], r0, sem0)
            pltpu.make_async_copy(r1, acc.at[idx_d.at[j + 1]], sem1).wait()
            pltpu.async_copy(y_hbm.at[idx_s.at[j + 3]], r1, sem1)

        j0 = HCH - 2
        j1 = HCH - 1
        pltpu.make_async_copy(y_hbm.at[idx_s.at[j0]], r0, sem0).wait()
        pltpu.async_copy(r0, acc.at[idx_d.at[j0]], sem0, add=True)
        pltpu.make_async_copy(y_hbm.at[idx_s.at[j1]], r1, sem1).wait()
        pltpu.async_copy(r1, acc.at[idx_d.at[j1]], sem1, add=True)
        pltpu.make_async_copy(r0, acc.at[idx_d.at[j0]], sem0).wait()
        pltpu.make_async_copy(r1, acc.at[idx_d.at[j1]], sem1).wait()

    plsc.subcore_barrier()
    pltpu.sync_copy(acc.at[sl], out_hbm.at[c].at[sl])


@functools.partial(jax.jit, static_argnames=("d",))
def _sc_agg(y, src_r, dst_r, zeros, *, d):
    f32 = jnp.float32
    return pl.kernel(
        _agg_body,
        out_type=jax.ShapeDtypeStruct((NC, N_P, d), f32),
        mesh=_MESH,
        scratch_types=[
            pltpu.VMEM((HCH, K), jnp.int32),
            pltpu.VMEM((HCH, K), jnp.int32),
            pltpu.VMEM((2, K, d), f32),
            pltpu.VMEM_SHARED((N_P, d), f32),
            pltpu.SemaphoreType.DMA,
            pltpu.SemaphoreType.DMA,
        ],
    )(y, src_r, dst_r, zeros)


_BLK = 1000
_HIGHEST = jax.lax.Precision.HIGHEST


def _tc_first_body(f_ref, hs0_ref, hs1_ref, w_ref, o_ref):
    deg = hs0_ref[:, :1] + hs1_ref[:, :1]
    ns = lax.rsqrt(jnp.maximum(deg, 1.0))
    o_ref[...] = jnp.dot(f_ref[...] * ns, w_ref[...],
                         preferred_element_type=jnp.float32,
                         precision=_HIGHEST)


def _tc_mid_body(a0_ref, a1_ref, hd0_ref, hd1_ref, hs0_ref, hs1_ref,
                 b_ref, w_ref, o_ref):
    degd = hd0_ref[:, :1] + hd1_ref[:, :1]
    nd = lax.rsqrt(jnp.maximum(degd, 1.0))
    degs = hs0_ref[:, :1] + hs1_ref[:, :1]
    ns = lax.rsqrt(jnp.maximum(degs, 1.0))
    h = jnp.maximum((a0_ref[...] + a1_ref[...]) * nd + b_ref[...], 0.0)
    o_ref[...] = jnp.dot(h * ns, w_ref[...],
                         preferred_element_type=jnp.float32,
                         precision=_HIGHEST)


def _tc_prefinal_body(a0_ref, a1_ref, hd0_ref, hd1_ref, hs0_ref, hs1_ref,
                      b_ref, o_ref):
    degd = hd0_ref[:, :1] + hd1_ref[:, :1]
    nd = lax.rsqrt(jnp.maximum(degd, 1.0))
    degs = hs0_ref[:, :1] + hs1_ref[:, :1]
    ns = lax.rsqrt(jnp.maximum(degs, 1.0))
    h = jnp.maximum((a0_ref[...] + a1_ref[...]) * nd + b_ref[...], 0.0)
    o_ref[...] = h * ns


def _tc_final_body(a0_ref, a1_ref, hd0_ref, hd1_ref, b_ref, w_ref, o_ref):
    degd = hd0_ref[:, :1] + hd1_ref[:, :1]
    nd = lax.rsqrt(jnp.maximum(degd, 1.0))
    agg = (a0_ref[...] + a1_ref[...]) * nd
    o_ref[...] = jnp.dot(agg, w_ref[...],
                         preferred_element_type=jnp.float32,
                         precision=_HIGHEST) + b_ref[...]


def _rows(d):
    return pl.BlockSpec((_BLK, d), lambda i: (i, 0))


def _whole(shape):
    return pl.BlockSpec(shape, lambda i: (0,) * len(shape))


def _tc_first(features, hs0, hs1, W):
    dout = W.shape[1]
    return pl.pallas_call(
        _tc_first_body,
        grid=(N // _BLK,),
        in_specs=[_rows(D_H), _rows(D_H), _rows(D_H), _whole((D_H, dout))],
        out_specs=_rows(dout),
        out_shape=jax.ShapeDtypeStruct((N, dout), jnp.float32),
    )(features, hs0, hs1, W)


def _tc_mid(a0, a1, hd0, hd1, hs0, hs1, b, W):
    din, dout = W.shape
    return pl.pallas_call(
        _tc_mid_body,
        grid=(N // _BLK,),
        in_specs=[_rows(din), _rows(din), _rows(D_H), _rows(D_H),
                  _rows(D_H), _rows(D_H), _whole((1, din)), _whole((din, dout))],
        out_specs=_rows(dout),
        out_shape=jax.ShapeDtypeStruct((N, dout), jnp.float32),
    )(a0, a1, hd0, hd1, hs0, hs1, b, W)


def _tc_prefinal(a0, a1, hd0, hd1, hs0, hs1, b):
    d = a0.shape[1]
    return pl.pallas_call(
        _tc_prefinal_body,
        grid=(N // _BLK,),
        in_specs=[_rows(d), _rows(d), _rows(D_H), _rows(D_H),
                  _rows(D_H), _rows(D_H), _whole((1, d))],
        out_specs=_rows(d),
        out_shape=jax.ShapeDtypeStruct((N, d), jnp.float32),
    )(a0, a1, hd0, hd1, hs0, hs1, b)


def _tc_final(a0, a1, hd0, hd1, b, W):
    din, dout = W.shape
    return pl.pallas_call(
        _tc_final_body,
        grid=(N // _BLK,),
        in_specs=[_rows(din), _rows(din), _rows(D_H), _rows(D_H),
                  _whole((1, dout)), _whole((din, dout))],
        out_specs=_rows(dout),
        out_shape=jax.ShapeDtypeStruct((N, dout), jnp.float32),
    )(a0, a1, hd0, hd1, b, W)


def kernel(features, edge_index, W1, b1, W2, b2, W3, b3):
    f32 = jnp.float32
    src_r = edge_index[0].reshape(NW, NCH, K)
    dst_r = edge_index[1].reshape(NW, NCH, K)
    zeros128 = jnp.zeros((N_P, D_H), f32)
    ones128 = jnp.ones((128, D_H), f32)

    hs, hd = _sc_hist(src_r, dst_r, zeros128, ones128)
    hs0, hs1 = hs[0], hs[1]
    hd0, hd1 = hd[0], hd[1]

    y1 = _tc_first(features, hs0, hs1, W1)
    a1 = _sc_agg(y1, src_r, dst_r, zeros128, d=D_H)
    y2 = _tc_mid(a1[0], a1[1], hd0, hd1, hs0, hs1, b1.reshape(1, D_H), W2)
    a2 = _sc_agg(y2, src_r, dst_r, zeros128, d=D_H)
    h2 = _tc_prefinal(a2[0], a2[1], hd0, hd1, hs0, hs1, b2.reshape(1, D_H))
    a3 = _sc_agg(h2, src_r, dst_r, zeros128, d=D_H)
    return _tc_final(a3[0], a3[1], hd0, hd1, b3.reshape(1, N_CLS), W3)
